# baseline (speedup 1.0000x reference)
import functools

import jax
import jax.numpy as jnp
from jax import lax
from jax.experimental import pallas as pl
from jax.experimental.pallas import tpu as pltpu
from jax.experimental.pallas import tpu_sc as plsc

_NUM_CORES = 2
_NUM_SUBCORES = 16
_NW = _NUM_CORES * _NUM_SUBCORES

_CHUNK = 256
_NBUF = 2


@functools.partial(jax.jit, static_argnums=(2, 3, 4))
def _gather_t(idx_t, table, seq, batch, h):
    b_per_w = batch // _NW
    n_chunks = seq * b_per_w // _CHUNK
    chunks_per_l = b_per_w // _CHUNK

    mesh = plsc.VectorSubcoreMesh(core_axis_name="c", subcore_axis_name="s")

    @functools.partial(
        pl.kernel,
        mesh=mesh,
        out_type=jax.ShapeDtypeStruct(
            (seq, h // 8, batch // 128, 8, 128), jnp.float32
        ),
        compiler_params=pltpu.CompilerParams(
            use_tc_tiling_on_sc=False, needs_layout_passes=False
        ),
        scratch_types=[
            pltpu.VMEM((seq, b_per_w), jnp.int32),
            pltpu.VMEM((_NBUF, _CHUNK, h), jnp.float32),
            pltpu.VMEM((_NBUF, h, _CHUNK), jnp.float32),
            pltpu.SemaphoreType.DMA((_NBUF,)),
            pltpu.SemaphoreType.DMA((_NBUF,)),
        ],
    )
    def body(idx_hbm, table_hbm, out_hbm, idx_v, rows_v, tbuf_v, gsem, wsem):
        wid = lax.axis_index("s") * _NUM_CORES + lax.axis_index("c")
        b0 = wid * b_per_w
        pltpu.sync_copy(idx_hbm.at[:, pl.ds(b0, b_per_w)], idx_v)

        lane = lax.iota(jnp.int32, 16)

        def fire_gather(l, off, s):
            pltpu.async_copy(
                table_hbm.at[idx_v.at[l, pl.ds(off, _CHUNK)]],
                rows_v.at[s],
                gsem.at[s],
            )

        def wait_gather(s):
            pltpu.make_async_copy(
                table_hbm.at[pl.ds(0, _CHUNK)], rows_v.at[s], gsem.at[s]
            ).wait()

        def fire_writeback(l, off, s):
            tc0 = (b0 + off) // 128
            for tr in range(h // 8):
                for jc in range(_CHUNK // 128):
                    pltpu.async_copy(
                        tbuf_v.at[s, pl.ds(tr * 8, 8), pl.ds(jc * 128, 128)],
                        out_hbm.at[l, tr, tc0 + jc, :, :],
                        wsem.at[s],
                    )

        def wait_writeback(s):
            for tr in range(h // 8):
                for jc in range(_CHUNK // 128):
                    pltpu.make_async_copy(
                        tbuf_v.at[s, pl.ds(tr * 8, 8), pl.ds(jc * 128, 128)],
                        out_hbm.at[0, 0, 0, :, :],
                        wsem.at[s],
                    ).wait()

        hvecs = [hb * 16 + lane for hb in range(h // 16)]

        def transpose_chunk(s):
            @plsc.parallel_loop(0, _CHUNK, unroll=4)
            def _(j):
                jvec = jnp.full((16,), j, jnp.int32)
                for hb in range(h // 16):
                    v = rows_v[s, j, pl.ds(hb * 16, 16)]
                    plsc.store_scatter(
                        tbuf_v.at[s], [hvecs[hb], jvec], v
                    )

        for s in range(_NBUF):
            fire_gather(s // chunks_per_l, (s % chunks_per_l) * _CHUNK, s)

        n_grp = n_chunks // _NBUF

        def outer(g, carry):
            for s in range(_NBUF):
                c = g * _NBUF + s
                l = c // chunks_per_l
                off = (c % chunks_per_l) * _CHUNK
                wait_gather(s)

                @pl.when(g > 0)
                def _():
                    wait_writeback(s)

                transpose_chunk(s)
                fire_writeback(l, off, s)

                @pl.when(g < n_grp - 1)
                def _():
                    nc = c + _NBUF
                    fire_gather(nc // chunks_per_l,
                                (nc % chunks_per_l) * _CHUNK, s)

            return carry

        lax.fori_loop(0, n_grp, outer, 0)
        for s in range(_NBUF):
            wait_writeback(s)

    return body(idx_t, table)


def kernel(indices, table):
    batch, seq = indices.shape
    _, h = table.shape
    idx_t = jnp.transpose(indices.astype(jnp.int32))
    out5 = _gather_t(idx_t, table, seq, batch, h)
    out = jnp.transpose(out5, (2, 4, 0, 1, 3))
    return out.reshape(batch, seq, h)


# speedup vs baseline: 1.7571x; 1.7571x over previous
import functools

import jax
import jax.numpy as jnp
from jax import lax
from jax.experimental import pallas as pl
from jax.experimental.pallas import tpu as pltpu
from jax.experimental.pallas import tpu_sc as plsc

_NUM_CORES = 2
_NUM_SUBCORES = 16
_NW = _NUM_CORES * _NUM_SUBCORES

_CHUNK = 256
_NBUF = 2


@functools.partial(jax.jit, static_argnums=(2, 3, 4))
def _gather_t(idx_t, table, seq, batch, h):
    b_per_w = batch // _NW
    n_chunks = seq * b_per_w // _CHUNK
    chunks_per_l = b_per_w // _CHUNK

    mesh = plsc.VectorSubcoreMesh(core_axis_name="c", subcore_axis_name="s")

    @functools.partial(
        pl.kernel,
        mesh=mesh,
        out_type=jax.ShapeDtypeStruct(
            (seq, h // 8, batch // 128, 8, 128), jnp.float32
        ),
        compiler_params=pltpu.CompilerParams(
            use_tc_tiling_on_sc=False, needs_layout_passes=False
        ),
        scratch_types=[
            pltpu.VMEM((seq, b_per_w), jnp.int32),
            pltpu.VMEM((_NBUF, _CHUNK, h), jnp.float32),
            pltpu.VMEM((_NBUF, h, _CHUNK), jnp.float32),
            pltpu.SemaphoreType.DMA((_NBUF,)),
            pltpu.SemaphoreType.DMA((_NBUF,)),
        ],
    )
    def body(idx_hbm, table_hbm, out_hbm, idx_v, rows_v, tbuf_v, gsem, wsem):
        wid = lax.axis_index("s") * _NUM_CORES + lax.axis_index("c")
        b0 = wid * b_per_w
        pltpu.sync_copy(idx_hbm.at[:, pl.ds(b0, b_per_w)], idx_v)

        lane = lax.iota(jnp.int32, 16)

        def fire_gather(l, off, s):
            pltpu.async_copy(
                table_hbm.at[idx_v.at[l, pl.ds(off, _CHUNK)]],
                rows_v.at[s],
                gsem.at[s],
            )

        def wait_gather(s):
            pltpu.make_async_copy(
                table_hbm.at[pl.ds(0, _CHUNK)], rows_v.at[s], gsem.at[s]
            ).wait()

        def fire_writeback(l, off, s):
            tc0 = (b0 + off) // 128
            for tr in range(h // 8):
                for jc in range(_CHUNK // 128):
                    pltpu.async_copy(
                        tbuf_v.at[s, pl.ds(tr * 8, 8), pl.ds(jc * 128, 128)],
                        out_hbm.at[l, tr, tc0 + jc, :, :],
                        wsem.at[s],
                    )

        def wait_writeback(s):
            for tr in range(h // 8):
                for jc in range(_CHUNK // 128):
                    pltpu.make_async_copy(
                        tbuf_v.at[s, pl.ds(tr * 8, 8), pl.ds(jc * 128, 128)],
                        out_hbm.at[0, 0, 0, :, :],
                        wsem.at[s],
                    ).wait()

        jvecs = [jb * 16 + lane for jb in range(_CHUNK // 16)]

        def transpose_chunk(s):
            @plsc.parallel_loop(0, h, unroll=2)
            def _(h0):
                hvec = (h0 + lane) & (h - 1)
                for jb in range(_CHUNK // 16):
                    v = plsc.load_gather(rows_v.at[s], [jvecs[jb], hvec])
                    plsc.store_scatter(tbuf_v.at[s], [hvec, jvecs[jb]], v)

        for s in range(_NBUF):
            fire_gather(s // chunks_per_l, (s % chunks_per_l) * _CHUNK, s)

        n_grp = n_chunks // _NBUF

        def outer(g, carry):
            for s in range(_NBUF):
                c = g * _NBUF + s
                l = c // chunks_per_l
                off = (c % chunks_per_l) * _CHUNK
                wait_gather(s)

                @pl.when(g > 0)
                def _():
                    wait_writeback(s)

                transpose_chunk(s)
                fire_writeback(l, off, s)

                @pl.when(g < n_grp - 1)
                def _():
                    nc = c + _NBUF
                    fire_gather(nc // chunks_per_l,
                                (nc % chunks_per_l) * _CHUNK, s)

            return carry

        lax.fori_loop(0, n_grp, outer, 0)
        for s in range(_NBUF):
            wait_writeback(s)

    return body(idx_t, table)


def kernel(indices, table):
    batch, seq = indices.shape
    _, h = table.shape
    idx_t = jnp.transpose(indices.astype(jnp.int32))
    out5 = _gather_t(idx_t, table, seq, batch, h)
    out = jnp.transpose(out5, (2, 4, 0, 1, 3))
    return out.reshape(batch, seq, h)
